# baseline (device time: 92803 ns/iter reference)
import jax
import jax.numpy as jnp
from jax import lax
from jax.experimental import pallas as pl
from jax.experimental.pallas import tpu as pltpu

N_DEV = 4
B = 2
SQ = 512
DM = 768
NH = 8
DH = 64
HD = NH * DH
WIN = 128
KVU = SQ + WIN
HALF = SQ // 2
QTR = SQ // 4

BF = jnp.bfloat16
F32 = jnp.float32


def kernel(x, Wq, K_ext, V_ext, Wo):
    k2 = K_ext.reshape(B, SQ, N_DEV * HD).astype(BF)
    v2 = V_ext.reshape(B, SQ, N_DEV * HD).astype(BF)

    def body(x_ref, wq_ref, k_ref, v_ref, wo_ref, out_ref,
             kg, vg,
             whsend, hrecv, wqsend, qrecv, ssend, agr1, h2send, agr2,
             ksend, vsend, kvrecv, locsem, csems, crecv):
        my = lax.axis_index("i")

        def kv_rdma(src, d, bb, hbm, vm, send_sems, t):
            if src == 0:
                s = hbm.at[bb, :, pl.ds(d * HD, HD)]
                dst = vm.at[bb, pl.ds(0, SQ), :]
            else:
                s = hbm.at[bb, pl.ds(0, WIN), pl.ds(d * HD, HD)]
                dst = vm.at[bb, pl.ds(SQ, WIN), :]
            return pltpu.make_async_remote_copy(
                src_ref=s, dst_ref=dst,
                send_sem=send_sems.at[d, bb], recv_sem=kvrecv.at[src, t, bb],
                device_id=(d,), device_id_type=pl.DeviceIdType.MESH,
            )

        def loc_copy(src, bb, hbm, vm, t):
            if src == 0:
                s = hbm.at[bb, :, pl.ds(src * HD, HD)]
                dst = vm.at[bb, pl.ds(0, SQ), :]
            else:
                s = hbm.at[bb, pl.ds(0, WIN), pl.ds(src * HD, HD)]
                dst = vm.at[bb, pl.ds(SQ, WIN), :]
            return pltpu.make_async_copy(s, dst, locsem.at[t, bb])

        for src in (0, 1):
            @pl.when(my == src)
            def _(src=src):
                for bb in range(B):
                    for d in range(N_DEV):
                        if d == src:
                            continue
                        kv_rdma(src, d, bb, k_ref, kg, ksend, 0).start()
                        kv_rdma(src, d, bb, v_ref, vg, vsend, 1).start()
                    loc_copy(src, bb, k_ref, kg, 0).start()
                    loc_copy(src, bb, v_ref, vg, 1).start()

        q_all = [
            jnp.dot(x_ref[b], wq_ref[...], preferred_element_type=F32)
            for b in range(B)
        ]

        i1 = lax.broadcasted_iota(jnp.int32, (HALF, 384), 0)
        j1 = lax.broadcasted_iota(jnp.int32, (HALF, 384), 1)
        mask1 = jnp.abs(i1 - j1) <= WIN
        i2 = lax.broadcasted_iota(jnp.int32, (HALF, SQ), 0)
        j2 = lax.broadcasted_iota(jnp.int32, (HALF, SQ), 1)
        mask2 = jnp.abs(i2 + WIN - j2) <= WIN

        def wait_kv(bb):
            for src in (0, 1):
                @pl.when(my == src)
                def _(src=src):
                    loc_copy(src, bb, k_ref, kg, 0).wait()
                    loc_copy(src, bb, v_ref, vg, 1).wait()

                @pl.when(my != src)
                def _(src=src):
                    kv_rdma(src, 0, bb, k_ref, kg, ksend, 0).wait_recv()
                    kv_rdma(src, 0, bb, v_ref, vg, vsend, 1).wait_recv()

        def sm_block(q, k, v, msk):
            s = lax.dot_general(
                q, k, (((1,), (1,)), ((), ())), preferred_element_type=F32,
            ) * 0.125
            s = jnp.where(msk, s, -1e9)
            m = jnp.max(s, axis=-1, keepdims=True)
            w = jnp.exp(s - m)
            w = w / jnp.sum(w, axis=-1, keepdims=True)
            return jnp.dot(w, v, preferred_element_type=F32)

        def attn_batch(bb):
            qb = q_all[bb].astype(BF)
            c1s, c2s = [], []
            for h in range(NH):
                cols = pl.ds(h * DH, DH)
                k1 = kg[bb, 0:384, cols]
                v1 = vg[bb, 0:384, cols].astype(F32)
                k2b = kg[bb, WIN:KVU, cols]
                v2b = vg[bb, WIN:KVU, cols].astype(F32)
                qh = qb[:, h * DH:(h + 1) * DH]
                c1s.append(sm_block(qh[0:HALF], k1, v1, mask1))
                c2s.append(sm_block(qh[HALF:SQ], k2b, v2b, mask2))
            c1 = jnp.concatenate(c1s, axis=1)
            c2 = jnp.concatenate(c2s, axis=1)
            return jnp.concatenate([
                jnp.dot(c1, wo_ref[...], preferred_element_type=F32),
                jnp.dot(c2, wo_ref[...], preferred_element_type=F32),
            ], axis=0)

        def xchg(src_ref_, dst_ref_, partner, step, bb):
            return pltpu.make_async_remote_copy(
                src_ref=src_ref_, dst_ref=dst_ref_,
                send_sem=csems.at[step, bb], recv_sem=crecv.at[step, bb],
                device_id=(partner,), device_id_type=pl.DeviceIdType.MESH,
            )

        def pos_params(pos):
            p1 = 3 - pos
            p2 = pos ^ 1
            a = pos >> 1
            bq = pos & 1
            mh = HALF * a
            oh = HALF * (1 - a)
            mq = mh + QTR * bq
            oq = mh + QTR * (1 - bq)
            return p1, p2, bq, mh, oh, mq, oq

        def seg_of(partial_b, bb, bq, mq):
            return (
                partial_b[mq:mq + QTR, :]
                + hrecv[bb, QTR * bq:QTR * bq + QTR, :].astype(F32)
                + qrecv[bb].astype(F32)
            )

        wait_kv(0)
        partial0 = attn_batch(0)
        for pos in range(N_DEV):
            p1, p2, bq, mh, oh, mq, oq = pos_params(pos)

            @pl.when(my == pos)
            def _(p1=p1, oh=oh):
                whsend[0] = partial0[oh:oh + HALF, :].astype(BF)
                xchg(whsend.at[0], hrecv.at[0], p1, 0, 0).start()

        for pos in range(N_DEV):
            p1, p2, bq, mh, oh, mq, oq = pos_params(pos)

            @pl.when(my == pos)
            def _(p1=p1, p2=p2, bq=bq, mh=mh):
                xchg(whsend.at[0], hrecv.at[0], p1, 0, 0).wait_recv()
                off = QTR * (1 - bq)
                wqsend[0] = (
                    partial0[mh + off:mh + off + QTR, :]
                    + hrecv[0, off:off + QTR, :].astype(F32)
                ).astype(BF)
                xchg(wqsend.at[0], qrecv.at[0], p2, 1, 0).start()

        wait_kv(1)
        partial1 = attn_batch(1)

        for pos in range(N_DEV):
            p1, p2, bq, mh, oh, mq, oq = pos_params(pos)

            @pl.when(my == pos)
            def _(p1=p1, p2=p2, bq=bq, mh=mh, oh=oh, mq=mq, oq=oq):
                whsend[1] = partial1[oh:oh + HALF, :].astype(BF)
                xchg(whsend.at[1], hrecv.at[1], p1, 0, 1).start()
                xchg(wqsend.at[0], qrecv.at[0], p2, 1, 0).wait_recv()
                seg0 = seg_of(partial0, 0, bq, mq)
                ssend[0] = seg0.astype(BF)
                xchg(ssend.at[0], agr1.at[0], p2, 2, 0).start()
                xchg(whsend.at[1], hrecv.at[1], p1, 0, 1).wait_recv()
                off = QTR * (1 - bq)
                wqsend[1] = (
                    partial1[mh + off:mh + off + QTR, :]
                    + hrecv[1, off:off + QTR, :].astype(F32)
                ).astype(BF)
                xchg(wqsend.at[1], qrecv.at[1], p2, 1, 1).start()
                xchg(ssend.at[0], agr1.at[0], p2, 2, 0).wait_recv()
                h2send[0, QTR * bq:QTR * bq + QTR, :] = ssend[0]
                h2send[0, off:off + QTR, :] = agr1[0]
                xchg(h2send.at[0], agr2.at[0], p1, 3, 0).start()
                out_ref[0, mq:mq + QTR, :] = seg0
                out_ref[0, oq:oq + QTR, :] = agr1[0].astype(F32)
                xchg(wqsend.at[1], qrecv.at[1], p2, 1, 1).wait_recv()
                seg1 = seg_of(partial1, 1, bq, mq)
                ssend[1] = seg1.astype(BF)
                xchg(ssend.at[1], agr1.at[1], p2, 2, 1).start()
                xchg(ssend.at[1], agr1.at[1], p2, 2, 1).wait_recv()
                h2send[1, QTR * bq:QTR * bq + QTR, :] = ssend[1]
                h2send[1, off:off + QTR, :] = agr1[1]
                xchg(h2send.at[1], agr2.at[1], p1, 3, 1).start()
                out_ref[1, mq:mq + QTR, :] = seg1
                out_ref[1, oq:oq + QTR, :] = agr1[1].astype(F32)
                xchg(h2send.at[0], agr2.at[0], p1, 3, 0).wait_recv()
                out_ref[0, oh:oh + HALF, :] = agr2[0].astype(F32)
                xchg(h2send.at[1], agr2.at[1], p1, 3, 1).wait_recv()
                out_ref[1, oh:oh + HALF, :] = agr2[1].astype(F32)

                for bb in range(B):
                    for step, (sref, dref, pp) in enumerate((
                        (whsend, hrecv, p1), (wqsend, qrecv, p2),
                        (ssend, agr1, p2), (h2send, agr2, p1),
                    )):
                        xchg(sref.at[bb], dref.at[bb], pp, step, bb).wait_send()

        for src in (0, 1):
            @pl.when(my == src)
            def _(src=src):
                for bb in range(B):
                    for d in range(N_DEV):
                        if d == src:
                            continue
                        kv_rdma(src, d, bb, k_ref, kg, ksend, 0).wait_send()
                        kv_rdma(src, d, bb, v_ref, vg, vsend, 1).wait_send()

    return pl.pallas_call(
        body,
        out_shape=jax.ShapeDtypeStruct((B, SQ, DM), F32),
        in_specs=[
            pl.BlockSpec(memory_space=pltpu.VMEM),
            pl.BlockSpec(memory_space=pltpu.VMEM),
            pl.BlockSpec(memory_space=pl.ANY),
            pl.BlockSpec(memory_space=pl.ANY),
            pl.BlockSpec(memory_space=pltpu.VMEM),
        ],
        out_specs=pl.BlockSpec(memory_space=pltpu.VMEM),
        scratch_shapes=[
            pltpu.VMEM((B, KVU, HD), BF),
            pltpu.VMEM((B, KVU, HD), BF),
            pltpu.VMEM((B, HALF, DM), BF),
            pltpu.VMEM((B, HALF, DM), BF),
            pltpu.VMEM((B, QTR, DM), BF),
            pltpu.VMEM((B, QTR, DM), BF),
            pltpu.VMEM((B, QTR, DM), BF),
            pltpu.VMEM((B, QTR, DM), BF),
            pltpu.VMEM((B, HALF, DM), BF),
            pltpu.VMEM((B, HALF, DM), BF),
            pltpu.SemaphoreType.DMA((N_DEV, B)),
            pltpu.SemaphoreType.DMA((N_DEV, B)),
            pltpu.SemaphoreType.DMA((2, 2, B)),
            pltpu.SemaphoreType.DMA((2, B)),
            pltpu.SemaphoreType.DMA((4, B)),
            pltpu.SemaphoreType.DMA((4, B)),
        ],
    )(x, Wq, k2, v2, Wo)
